# baseline (speedup 1.0000x reference)
import functools

import jax
import jax.numpy as jnp
from jax import lax
from jax.experimental import pallas as pl
from jax.experimental.pallas import tpu as pltpu
from jax.experimental.pallas import tpu_sc as plsc

N_E = 1024
E_DIM = 256
N_TOK = 16384
TOK_BLK = 1024
N_BLK = N_TOK // TOK_BLK

_SC_CORES = 2
_SC_SUBCORES = 16
_NW = _SC_CORES * _SC_SUBCORES
_B_PER_W = N_TOK // _NW
_CHUNK = 128
_N_CHUNK = _B_PER_W // _CHUNK


def _dist_argmin_body(z_ref, cb_ref, idx_ref, loss_ref):
    i = pl.program_id(0)
    zblk = z_ref[...]
    cb = cb_ref[...]
    m = lax.dot_general(zblk, cb, (((1,), (1,)), ((), ())),
                        preferred_element_type=jnp.float32)
    a = jnp.sum(zblk * zblk, axis=1, keepdims=True)
    b = jnp.sum(cb * cb, axis=1)[None, :]
    d = (a + b) - 2.0 * m
    mind = jnp.min(d, axis=1, keepdims=True)
    jidx = lax.broadcasted_iota(jnp.int32, d.shape, 1)
    idx = jnp.min(jnp.where(d == mind, jidx, N_E), axis=1)
    idx_ref[0, 0, :] = idx

    @pl.when(i == 0)
    def _init():
        loss_ref[0, 0] = 0.0

    loss_ref[0, 0] += jnp.sum(mind)

    @pl.when(i == N_BLK - 1)
    def _finish():
        loss_ref[0, 0] = loss_ref[0, 0] * (1.25 / (N_TOK * E_DIM))


def _dist_argmin(z_flat, codebook):
    return pl.pallas_call(
        _dist_argmin_body,
        grid=(N_BLK,),
        in_specs=[
            pl.BlockSpec((TOK_BLK, E_DIM), lambda i: (i, 0)),
            pl.BlockSpec((N_E, E_DIM), lambda i: (0, 0)),
        ],
        out_specs=[
            pl.BlockSpec((1, 1, TOK_BLK), lambda i: (i, 0, 0)),
            pl.BlockSpec(block_shape=(1, 1), index_map=lambda i: (0, 0),
                         memory_space=pltpu.SMEM),
        ],
        out_shape=[
            jax.ShapeDtypeStruct((N_BLK, 1, TOK_BLK), jnp.int32),
            jax.ShapeDtypeStruct((1, 1), jnp.float32),
        ],
    )(z_flat, codebook)


@functools.cache
def _gather_fn():
    mesh = plsc.VectorSubcoreMesh(core_axis_name="c", subcore_axis_name="s")

    @functools.partial(
        pl.kernel, mesh=mesh,
        out_type=jax.ShapeDtypeStruct((N_TOK, E_DIM), jnp.float32),
        scratch_types=[
            pltpu.VMEM((_B_PER_W,), jnp.int32),
            pltpu.VMEM((_CHUNK, E_DIM), jnp.float32),
            pltpu.VMEM((_CHUNK, E_DIM), jnp.float32),
            pltpu.SemaphoreType.DMA,
            pltpu.SemaphoreType.DMA,
        ],
    )
    def gather(cb_hbm, idx_hbm, out_hbm, idx_v, buf0, buf1, sem0, sem1):
        wid = lax.axis_index("s") * _SC_CORES + lax.axis_index("c")
        base = wid * _B_PER_W
        pltpu.sync_copy(idx_hbm.at[pl.ds(base, _B_PER_W)], idx_v)
        bufs = (buf0, buf1)
        sems = (sem0, sem1)
        copies = []
        for k in range(_N_CHUNK):
            copies.append(pltpu.async_copy(
                cb_hbm.at[idx_v.at[pl.ds(k * _CHUNK, _CHUNK)]],
                bufs[k % 2], sems[k % 2]))
            if k >= 1:
                copies[k - 1].wait()
                pltpu.sync_copy(bufs[(k - 1) % 2],
                                out_hbm.at[pl.ds(base + (k - 1) * _CHUNK, _CHUNK)])
        copies[-1].wait()
        pltpu.sync_copy(bufs[(_N_CHUNK - 1) % 2],
                        out_hbm.at[pl.ds(base + (_N_CHUNK - 1) * _CHUNK, _CHUNK)])

    return gather


def kernel(z, codebook):
    zp = jnp.transpose(z, (0, 2, 3, 1))
    z_flat = zp.reshape(N_TOK, E_DIM)
    idx3, loss11 = _dist_argmin(z_flat, codebook)
    idx = idx3.reshape(N_TOK)
    zq_flat = _gather_fn()(codebook, idx)
    z_q_out = jnp.transpose(zq_flat.reshape(16, 32, 32, E_DIM), (0, 3, 1, 2))
    return z_q_out, loss11[0, 0], idx


# speedup vs baseline: 1.0595x; 1.0595x over previous
import functools

import jax
import jax.numpy as jnp
from jax import lax
from jax.experimental import pallas as pl
from jax.experimental.pallas import tpu as pltpu
from jax.experimental.pallas import tpu_sc as plsc

N_E = 1024
E_DIM = 256
N_TOK = 16384
TOK_BLK = 1024
N_BLK = N_TOK // TOK_BLK

_SC_CORES = 2
_SC_SUBCORES = 16
_NW = _SC_CORES * _SC_SUBCORES
_B_PER_W = N_TOK // _NW
_CHUNK = 128
_N_CHUNK = _B_PER_W // _CHUNK


def _dist_argmin_body(z_ref, cb_ref, idx_ref, loss_ref):
    i = pl.program_id(0)
    zb = z_ref[0]
    cb = cb_ref[...]
    m = lax.dot_general(cb, zb, (((1,), (0,)), ((), ())),
                        preferred_element_type=jnp.float32)
    a = jnp.sum(zb * zb, axis=0, keepdims=True)
    b = jnp.sum(cb * cb, axis=1)[:, None]
    d = (a + b) - 2.0 * m
    mind = jnp.min(d, axis=0, keepdims=True)
    jidx = lax.broadcasted_iota(jnp.int32, d.shape, 0)
    idx = jnp.min(jnp.where(d == mind, jidx, N_E), axis=0)
    idx_ref[0, 0, :] = idx

    @pl.when(i == 0)
    def _init():
        loss_ref[0, 0] = 0.0

    loss_ref[0, 0] += jnp.sum(mind)

    @pl.when(i == N_BLK - 1)
    def _finish():
        loss_ref[0, 0] = loss_ref[0, 0] * (1.25 / (N_TOK * E_DIM))


def _dist_argmin(z_chw, codebook):
    return pl.pallas_call(
        _dist_argmin_body,
        grid=(N_BLK,),
        in_specs=[
            pl.BlockSpec((1, E_DIM, TOK_BLK), lambda i: (i, 0, 0)),
            pl.BlockSpec((N_E, E_DIM), lambda i: (0, 0)),
        ],
        out_specs=[
            pl.BlockSpec((1, 1, TOK_BLK), lambda i: (i, 0, 0)),
            pl.BlockSpec(block_shape=(1, 1), index_map=lambda i: (0, 0),
                         memory_space=pltpu.SMEM),
        ],
        out_shape=[
            jax.ShapeDtypeStruct((N_BLK, 1, TOK_BLK), jnp.int32),
            jax.ShapeDtypeStruct((1, 1), jnp.float32),
        ],
    )(z_chw, codebook)


@functools.cache
def _gather_fn():
    mesh = plsc.VectorSubcoreMesh(core_axis_name="c", subcore_axis_name="s")

    @functools.partial(
        pl.kernel, mesh=mesh,
        out_type=jax.ShapeDtypeStruct((N_TOK, E_DIM), jnp.float32),
        scratch_types=[
            pltpu.VMEM((_B_PER_W,), jnp.int32),
            pltpu.VMEM((_CHUNK, E_DIM), jnp.float32),
            pltpu.VMEM((_CHUNK, E_DIM), jnp.float32),
            pltpu.SemaphoreType.DMA,
            pltpu.SemaphoreType.DMA,
        ],
    )
    def gather(cb_hbm, idx_hbm, out_hbm, idx_v, buf0, buf1, sem0, sem1):
        wid = lax.axis_index("s") * _SC_CORES + lax.axis_index("c")
        base = wid * _B_PER_W
        pltpu.sync_copy(idx_hbm.at[pl.ds(base, _B_PER_W)], idx_v)
        bufs = (buf0, buf1)
        sems = (sem0, sem1)
        copies = []
        for k in range(_N_CHUNK):
            copies.append(pltpu.async_copy(
                cb_hbm.at[idx_v.at[pl.ds(k * _CHUNK, _CHUNK)]],
                bufs[k % 2], sems[k % 2]))
            if k >= 1:
                copies[k - 1].wait()
                pltpu.sync_copy(bufs[(k - 1) % 2],
                                out_hbm.at[pl.ds(base + (k - 1) * _CHUNK, _CHUNK)])
        copies[-1].wait()
        pltpu.sync_copy(bufs[(_N_CHUNK - 1) % 2],
                        out_hbm.at[pl.ds(base + (_N_CHUNK - 1) * _CHUNK, _CHUNK)])

    return gather


def kernel(z, codebook):
    z_chw = z.reshape(16, E_DIM, TOK_BLK)
    idx3, loss11 = _dist_argmin(z_chw, codebook)
    idx = idx3.reshape(N_TOK)
    zq_flat = _gather_fn()(codebook, idx)
    z_q_out = jnp.transpose(zq_flat.reshape(16, 32, 32, E_DIM), (0, 3, 1, 2))
    return z_q_out, loss11[0, 0], idx


# speedup vs baseline: 1.1080x; 1.0458x over previous
import functools

import jax
import jax.numpy as jnp
from jax import lax
from jax.experimental import pallas as pl
from jax.experimental.pallas import tpu as pltpu
from jax.experimental.pallas import tpu_sc as plsc

N_E = 1024
E_DIM = 256
N_TOK = 16384
TOK_BLK = 1024
N_BLK = N_TOK // TOK_BLK

_SC_CORES = 2
_SC_SUBCORES = 16
_NW = _SC_CORES * _SC_SUBCORES
_B_PER_W = N_TOK // _NW
_CHUNK = 128
_N_CHUNK = _B_PER_W // _CHUNK


_ROWS_PER_VREG = 8


def _dist_argmin_body(z_ref, cb2_ref, idx_ref, loss_ref):
    i = pl.program_id(0)
    zb = z_ref[0]
    cb2 = cb2_ref[...]
    m2 = lax.dot_general(cb2, zb, (((1,), (0,)), ((), ())),
                         preferred_element_type=jnp.float32)
    a = jnp.sum(zb * zb, axis=0, keepdims=True)
    b = 0.25 * jnp.sum(cb2 * cb2, axis=1)[:, None]
    d = (a + b) - m2
    ds3 = d.reshape(N_E // _ROWS_PER_VREG, _ROWS_PER_VREG, TOK_BLK)
    mv = ds3[0]
    mi = jnp.zeros((_ROWS_PER_VREG, TOK_BLK), jnp.int32)
    for r in range(1, N_E // _ROWS_PER_VREG):
        row = ds3[r]
        lt = row < mv
        mv = jnp.where(lt, row, mv)
        mi = jnp.where(lt, r, mi)
    srow = lax.broadcasted_iota(jnp.int32, (_ROWS_PER_VREG, TOK_BLK), 0)
    gi = mi * _ROWS_PER_VREG + srow
    mind = jnp.min(mv, axis=0, keepdims=True)
    idx = jnp.min(jnp.where(mv == mind, gi, N_E), axis=0)
    idx_ref[0, 0, :] = idx

    @pl.when(i == 0)
    def _init():
        loss_ref[0, 0] = 0.0

    loss_ref[0, 0] += jnp.sum(mind)

    @pl.when(i == N_BLK - 1)
    def _finish():
        loss_ref[0, 0] = loss_ref[0, 0] * (1.25 / (N_TOK * E_DIM))


def _dist_argmin(z_chw, codebook):
    return pl.pallas_call(
        _dist_argmin_body,
        grid=(N_BLK,),
        in_specs=[
            pl.BlockSpec((1, E_DIM, TOK_BLK), lambda i: (i, 0, 0)),
            pl.BlockSpec((N_E, E_DIM), lambda i: (0, 0)),
        ],
        out_specs=[
            pl.BlockSpec((1, 1, TOK_BLK), lambda i: (i, 0, 0)),
            pl.BlockSpec(block_shape=(1, 1), index_map=lambda i: (0, 0),
                         memory_space=pltpu.SMEM),
        ],
        out_shape=[
            jax.ShapeDtypeStruct((N_BLK, 1, TOK_BLK), jnp.int32),
            jax.ShapeDtypeStruct((1, 1), jnp.float32),
        ],
    )(z_chw, codebook)


@functools.cache
def _gather_fn():
    mesh = plsc.VectorSubcoreMesh(core_axis_name="c", subcore_axis_name="s")

    @functools.partial(
        pl.kernel, mesh=mesh,
        out_type=jax.ShapeDtypeStruct((N_TOK, E_DIM), jnp.float32),
        scratch_types=[
            pltpu.VMEM((_B_PER_W,), jnp.int32),
            pltpu.VMEM((_CHUNK, E_DIM), jnp.float32),
            pltpu.VMEM((_CHUNK, E_DIM), jnp.float32),
            pltpu.SemaphoreType.DMA,
            pltpu.SemaphoreType.DMA,
        ],
    )
    def gather(cb_hbm, idx_hbm, out_hbm, idx_v, buf0, buf1, sem0, sem1):
        wid = lax.axis_index("s") * _SC_CORES + lax.axis_index("c")
        base = wid * _B_PER_W
        pltpu.sync_copy(idx_hbm.at[pl.ds(base, _B_PER_W)], idx_v)
        bufs = (buf0, buf1)
        sems = (sem0, sem1)
        copies = []
        for k in range(_N_CHUNK):
            copies.append(pltpu.async_copy(
                cb_hbm.at[idx_v.at[pl.ds(k * _CHUNK, _CHUNK)]],
                bufs[k % 2], sems[k % 2]))
            if k >= 1:
                copies[k - 1].wait()
                pltpu.sync_copy(bufs[(k - 1) % 2],
                                out_hbm.at[pl.ds(base + (k - 1) * _CHUNK, _CHUNK)])
        copies[-1].wait()
        pltpu.sync_copy(bufs[(_N_CHUNK - 1) % 2],
                        out_hbm.at[pl.ds(base + (_N_CHUNK - 1) * _CHUNK, _CHUNK)])

    return gather


def kernel(z, codebook):
    z_chw = z.reshape(16, E_DIM, TOK_BLK)
    idx3, loss11 = _dist_argmin(z_chw, codebook + codebook)
    idx = idx3.reshape(N_TOK)
    zq_flat = _gather_fn()(codebook, idx)
    z_q_out = jnp.transpose(zq_flat.reshape(16, 32, 32, E_DIM), (0, 3, 1, 2))
    return z_q_out, loss11[0, 0], idx


# speedup vs baseline: 1.2793x; 1.1546x over previous
import functools

import jax
import jax.numpy as jnp
from jax import lax
from jax.experimental import pallas as pl
from jax.experimental.pallas import tpu as pltpu
from jax.experimental.pallas import tpu_sc as plsc

N_E = 1024
E_DIM = 256
N_TOK = 16384
TOK_BLK = 1024
N_BLK = N_TOK // TOK_BLK

_SC_CORES = 2
_SC_SUBCORES = 16
_NW = _SC_CORES * _SC_SUBCORES
_B_PER_W = N_TOK // _NW
_CHUNK = 128
_N_CHUNK = _B_PER_W // _CHUNK


_ROWS_PER_VREG = 8


def _dist_argmin_body(z_ref, cb_ref, idx_ref, loss_ref, b_ref):
    i = pl.program_id(0)
    zblk = z_ref[...]
    cb = cb_ref[...]
    cb2 = cb + cb
    m2 = lax.dot_general(cb2, zblk, (((1,), (1,)), ((), ())),
                         preferred_element_type=jnp.float32)
    zsq = zblk * zblk
    ones_row = jnp.ones((1, E_DIM), jnp.float32)
    a = lax.dot_general(ones_row, zsq, (((1,), (1,)), ((), ())),
                        preferred_element_type=jnp.float32)

    @pl.when(i == 0)
    def _precompute_b():
        b_ref[...] = jnp.sum(cb * cb, axis=1)[:, None]

    b = b_ref[...]
    d = (a + b) - m2
    ds3 = d.reshape(N_E // _ROWS_PER_VREG, _ROWS_PER_VREG, TOK_BLK)
    mv = ds3[0]
    mi = jnp.zeros((_ROWS_PER_VREG, TOK_BLK), jnp.int32)
    for r in range(1, N_E // _ROWS_PER_VREG):
        row = ds3[r]
        lt = row < mv
        mv = jnp.where(lt, row, mv)
        mi = jnp.where(lt, r, mi)
    srow = lax.broadcasted_iota(jnp.int32, (_ROWS_PER_VREG, TOK_BLK), 0)
    gi = mi * _ROWS_PER_VREG + srow
    mind = jnp.min(mv, axis=0, keepdims=True)
    idx = jnp.min(jnp.where(mv == mind, gi, N_E), axis=0)
    idx_ref[0, 0, :] = idx

    @pl.when(i == 0)
    def _init():
        loss_ref[0, 0] = 0.0

    loss_ref[0, 0] += jnp.sum(mind)

    @pl.when(i == N_BLK - 1)
    def _finish():
        loss_ref[0, 0] = loss_ref[0, 0] * (1.25 / (N_TOK * E_DIM))


def _dist_argmin(z_flat, codebook):
    return pl.pallas_call(
        _dist_argmin_body,
        grid=(N_BLK,),
        in_specs=[
            pl.BlockSpec((TOK_BLK, E_DIM), lambda i: (i, 0)),
            pl.BlockSpec((N_E, E_DIM), lambda i: (0, 0)),
        ],
        out_specs=[
            pl.BlockSpec((1, 1, TOK_BLK), lambda i: (i, 0, 0)),
            pl.BlockSpec(block_shape=(1, 1), index_map=lambda i: (0, 0),
                         memory_space=pltpu.SMEM),
        ],
        out_shape=[
            jax.ShapeDtypeStruct((N_BLK, 1, TOK_BLK), jnp.int32),
            jax.ShapeDtypeStruct((1, 1), jnp.float32),
        ],
        scratch_shapes=[pltpu.VMEM((N_E, 1), jnp.float32)],
    )(z_flat, codebook)


@functools.cache
def _gather_fn():
    mesh = plsc.VectorSubcoreMesh(core_axis_name="c", subcore_axis_name="s")

    @functools.partial(
        pl.kernel, mesh=mesh,
        out_type=jax.ShapeDtypeStruct((N_TOK, E_DIM), jnp.float32),
        scratch_types=[
            pltpu.VMEM((_B_PER_W,), jnp.int32),
            pltpu.VMEM((_CHUNK, E_DIM), jnp.float32),
            pltpu.VMEM((_CHUNK, E_DIM), jnp.float32),
            pltpu.SemaphoreType.DMA,
            pltpu.SemaphoreType.DMA,
        ],
    )
    def gather(cb_hbm, idx_hbm, out_hbm, idx_v, buf0, buf1, sem0, sem1):
        wid = lax.axis_index("s") * _SC_CORES + lax.axis_index("c")
        base = wid * _B_PER_W
        pltpu.sync_copy(idx_hbm.at[pl.ds(base, _B_PER_W)], idx_v)
        bufs = (buf0, buf1)
        sems = (sem0, sem1)
        copies = []
        for k in range(_N_CHUNK):
            copies.append(pltpu.async_copy(
                cb_hbm.at[idx_v.at[pl.ds(k * _CHUNK, _CHUNK)]],
                bufs[k % 2], sems[k % 2]))
            if k >= 1:
                copies[k - 1].wait()
                pltpu.sync_copy(bufs[(k - 1) % 2],
                                out_hbm.at[pl.ds(base + (k - 1) * _CHUNK, _CHUNK)])
        copies[-1].wait()
        pltpu.sync_copy(bufs[(_N_CHUNK - 1) % 2],
                        out_hbm.at[pl.ds(base + (_N_CHUNK - 1) * _CHUNK, _CHUNK)])

    return gather


def kernel(z, codebook):
    z_flat = jnp.transpose(z, (0, 2, 3, 1)).reshape(N_TOK, E_DIM)
    idx3, loss11 = _dist_argmin(z_flat, codebook)
    idx = idx3.reshape(N_TOK)
    zq_flat = _gather_fn()(codebook, idx)
    z_q_out = jnp.transpose(zq_flat.reshape(16, 32, 32, E_DIM), (0, 3, 1, 2))
    return z_q_out, loss11[0, 0], idx


# speedup vs baseline: 1.2996x; 1.0158x over previous
import functools

import jax
import jax.numpy as jnp
from jax import lax
from jax.experimental import pallas as pl
from jax.experimental.pallas import tpu as pltpu
from jax.experimental.pallas import tpu_sc as plsc

N_E = 1024
E_DIM = 256
N_TOK = 16384
TOK_BLK = 1024
N_BLK = N_TOK // TOK_BLK

_SC_CORES = 2
_SC_SUBCORES = 16
_NW = _SC_CORES * _SC_SUBCORES
_B_PER_W = N_TOK // _NW
_CHUNK = 128
_N_CHUNK = _B_PER_W // _CHUNK


_ROWS_PER_VREG = 8


def _dist_argmin_body(z_ref, cb_ref, idx_ref, loss_ref, lhs_ref):
    i = pl.program_id(0)
    zblk = z_ref[...]
    cb = cb_ref[...]
    cb2 = cb + cb
    m2 = lax.dot_general(cb2, zblk, (((1,), (1,)), ((), ())),
                         preferred_element_type=jnp.float32)
    zsq = zblk * zblk
    ones_row = jnp.ones((1, E_DIM), jnp.float32)
    a = lax.dot_general(ones_row, zsq, (((1,), (1,)), ((), ())),
                        preferred_element_type=jnp.float32)

    @pl.when(i == 0)
    def _precompute_b():
        b = jnp.sum(cb * cb, axis=1)[:, None]
        lhs_ref[...] = jnp.concatenate(
            [jnp.ones((N_E, 1), jnp.float32), b], axis=1)

    rhs = jnp.concatenate([a, jnp.ones((1, TOK_BLK), jnp.float32)], axis=0)
    t1 = lax.dot_general(lhs_ref[...], rhs, (((1,), (0,)), ((), ())),
                         preferred_element_type=jnp.float32)
    d = t1 - m2
    ds3 = d.reshape(N_E // _ROWS_PER_VREG, _ROWS_PER_VREG, TOK_BLK)
    mv = ds3[0]
    mi = jnp.zeros((_ROWS_PER_VREG, TOK_BLK), jnp.int32)
    for r in range(1, N_E // _ROWS_PER_VREG):
        row = ds3[r]
        lt = row < mv
        mv = jnp.where(lt, row, mv)
        mi = jnp.where(lt, r, mi)
    srow = lax.broadcasted_iota(jnp.int32, (_ROWS_PER_VREG, TOK_BLK), 0)
    gi = mi * _ROWS_PER_VREG + srow
    mind = jnp.min(mv, axis=0, keepdims=True)
    idx = jnp.min(jnp.where(mv == mind, gi, N_E), axis=0)
    idx_ref[0, 0, :] = idx

    @pl.when(i == 0)
    def _init():
        loss_ref[0, 0] = 0.0

    loss_ref[0, 0] += jnp.sum(mind)

    @pl.when(i == N_BLK - 1)
    def _finish():
        loss_ref[0, 0] = loss_ref[0, 0] * (1.25 / (N_TOK * E_DIM))


def _dist_argmin(z_flat, codebook):
    return pl.pallas_call(
        _dist_argmin_body,
        grid=(N_BLK,),
        in_specs=[
            pl.BlockSpec((TOK_BLK, E_DIM), lambda i: (i, 0)),
            pl.BlockSpec((N_E, E_DIM), lambda i: (0, 0)),
        ],
        out_specs=[
            pl.BlockSpec((1, 1, TOK_BLK), lambda i: (i, 0, 0)),
            pl.BlockSpec(block_shape=(1, 1), index_map=lambda i: (0, 0),
                         memory_space=pltpu.SMEM),
        ],
        out_shape=[
            jax.ShapeDtypeStruct((N_BLK, 1, TOK_BLK), jnp.int32),
            jax.ShapeDtypeStruct((1, 1), jnp.float32),
        ],
        scratch_shapes=[pltpu.VMEM((N_E, 2), jnp.float32)],
    )(z_flat, codebook)


@functools.cache
def _gather_fn():
    mesh = plsc.VectorSubcoreMesh(core_axis_name="c", subcore_axis_name="s")

    @functools.partial(
        pl.kernel, mesh=mesh,
        out_type=jax.ShapeDtypeStruct((N_TOK, E_DIM), jnp.float32),
        scratch_types=[
            pltpu.VMEM((_B_PER_W,), jnp.int32),
            pltpu.VMEM((_CHUNK, E_DIM), jnp.float32),
            pltpu.VMEM((_CHUNK, E_DIM), jnp.float32),
            pltpu.SemaphoreType.DMA,
            pltpu.SemaphoreType.DMA,
        ],
    )
    def gather(cb_hbm, idx_hbm, out_hbm, idx_v, buf0, buf1, sem0, sem1):
        wid = lax.axis_index("s") * _SC_CORES + lax.axis_index("c")
        base = wid * _B_PER_W
        pltpu.sync_copy(idx_hbm.at[pl.ds(base, _B_PER_W)], idx_v)
        bufs = (buf0, buf1)
        sems = (sem0, sem1)
        copies = []
        for k in range(_N_CHUNK):
            copies.append(pltpu.async_copy(
                cb_hbm.at[idx_v.at[pl.ds(k * _CHUNK, _CHUNK)]],
                bufs[k % 2], sems[k % 2]))
            if k >= 1:
                copies[k - 1].wait()
                pltpu.sync_copy(bufs[(k - 1) % 2],
                                out_hbm.at[pl.ds(base + (k - 1) * _CHUNK, _CHUNK)])
        copies[-1].wait()
        pltpu.sync_copy(bufs[(_N_CHUNK - 1) % 2],
                        out_hbm.at[pl.ds(base + (_N_CHUNK - 1) * _CHUNK, _CHUNK)])

    return gather


def kernel(z, codebook):
    z_flat = jnp.transpose(z, (0, 2, 3, 1)).reshape(N_TOK, E_DIM)
    idx3, loss11 = _dist_argmin(z_flat, codebook)
    idx = idx3.reshape(N_TOK)
    zq_flat = _gather_fn()(codebook, idx)
    z_q_out = jnp.transpose(zq_flat.reshape(16, 32, 32, E_DIM), (0, 3, 1, 2))
    return z_q_out, loss11[0, 0], idx
